# baseline (speedup 1.0000x reference)
import functools

import jax
import jax.numpy as jnp
from jax import lax
from jax.experimental import pallas as pl
from jax.experimental.pallas import tpu as pltpu
from jax.experimental.pallas import tpu_sc as plsc

VOCAB = 1000000
D = 64
HIDDEN = 84
OUT_D = 64
BATCH = 16384
HIST = 200

NC = 2
NS = 16
NW = NC * NS
S_PER_W = BATCH // NW
SB = 64
C = 8
T = HIST // C


DT_CHUNK = 400
NCH = VOCAB // DT_CHUNK
DT_ITERS = -(-NCH // NW)


def _detile_sc(emb):
    mesh = plsc.VectorSubcoreMesh(core_axis_name="c", subcore_axis_name="s")

    @functools.partial(
        pl.kernel,
        mesh=mesh,
        out_type=jax.ShapeDtypeStruct((VOCAB * D,), jnp.float32),
        scratch_types=[
            pltpu.VMEM((DT_CHUNK, D), jnp.float32),
            pltpu.VMEM((DT_CHUNK * D,), jnp.float32),
            pltpu.SemaphoreType.DMA,
        ],
    )
    def detile(emb_hbm, out_hbm, in_v, den_v, sem):
        wid = lax.axis_index("s") * NC + lax.axis_index("c")

        def chunk_body(k, carry):
            cid = wid + NW * k

            @pl.when(cid < NCH)
            def _():
                r0 = cid * DT_CHUNK
                pltpu.sync_copy(emb_hbm.at[pl.ds(r0, DT_CHUNK)], in_v)

                def rbody(r, carry2):
                    for u in range(2):
                        rr = 2 * r + u
                        den_v[pl.ds(rr * D + 0, 16)] = in_v[rr, pl.ds(0, 16)]
                        den_v[pl.ds(rr * D + 16, 16)] = in_v[rr, pl.ds(16, 16)]
                        den_v[pl.ds(rr * D + 32, 16)] = in_v[rr, pl.ds(32, 16)]
                        den_v[pl.ds(rr * D + 48, 16)] = in_v[rr, pl.ds(48, 16)]
                    return carry2

                lax.fori_loop(0, DT_CHUNK // 2, rbody, 0)
                pltpu.sync_copy(den_v, out_hbm.at[pl.ds(r0 * D, DT_CHUNK * D)])

            return carry

        lax.fori_loop(0, DT_ITERS, chunk_body, 0)

    return detile(emb)


def _pool_sc(x, emb_lin):
    x1 = x.reshape(BATCH * HIST)
    mesh = plsc.VectorSubcoreMesh(core_axis_name="c", subcore_axis_name="s")

    @functools.partial(
        pl.kernel,
        mesh=mesh,
        compiler_params=pltpu.CompilerParams(use_tc_tiling_on_sc=False),
        out_type=jax.ShapeDtypeStruct((BATCH * D,), jnp.float32),
        scratch_types=[
            pltpu.VMEM((SB * HIST,), jnp.int32),
            pltpu.VMEM((C, D), jnp.float32),
            pltpu.VMEM((C, D), jnp.float32),
            pltpu.VMEM((C, D), jnp.float32),
            pltpu.VMEM((C, D), jnp.float32),
            pltpu.VMEM((SB * D,), jnp.float32),
            pltpu.SemaphoreType.DMA,
            pltpu.SemaphoreType.DMA,
            pltpu.SemaphoreType.DMA,
            pltpu.SemaphoreType.DMA,
        ],
    )
    def pool(x_hbm, emb_hbm, out_hbm, idx_v, acc0, acc1, acc2, acc3,
             ostage_v, sem0, sem1, sem2, sem3):
        accs_sems = ((acc0, sem0), (acc1, sem1), (acc2, sem2), (acc3, sem3))
        wid = lax.axis_index("s") * NC + lax.axis_index("c")
        base = wid * S_PER_W
        inv = jnp.float32(1.0 / HIST)
        z = jnp.zeros((16,), jnp.float32)

        def zero(acc):
            def zbody(j, carry):
                for u in range(2):
                    acc[2 * j + u, pl.ds(0, 16)] = z
                    acc[2 * j + u, pl.ds(16, 16)] = z
                    acc[2 * j + u, pl.ds(32, 16)] = z
                    acc[2 * j + u, pl.ds(48, 16)] = z
                return carry
            lax.fori_loop(0, C // 2, zbody, 0)

        def fire(s, acc, sem):
            for t in range(T):
                pltpu.async_copy(
                    emb_hbm.at[idx_v.at[pl.ds(s * HIST + t * C, C)]],
                    acc, sem, add=True)

        def wait_all(acc, sem):
            cp = pltpu.make_async_copy(
                emb_hbm.at[idx_v.at[pl.ds(0, C)]], acc, sem)
            for t in range(T):
                cp.wait()

        def reduce_and_rezero(s, acc):
            def rbody(j, accs):
                a0, a1, a2, a3 = accs
                for u in range(2):
                    a0 = a0 + acc[2 * j + u, pl.ds(0, 16)]
                    a1 = a1 + acc[2 * j + u, pl.ds(16, 16)]
                    a2 = a2 + acc[2 * j + u, pl.ds(32, 16)]
                    a3 = a3 + acc[2 * j + u, pl.ds(48, 16)]
                    acc[2 * j + u, pl.ds(0, 16)] = z
                    acc[2 * j + u, pl.ds(16, 16)] = z
                    acc[2 * j + u, pl.ds(32, 16)] = z
                    acc[2 * j + u, pl.ds(48, 16)] = z
                return (a0, a1, a2, a3)
            a0, a1, a2, a3 = lax.fori_loop(0, C // 2, rbody, (z, z, z, z))
            ostage_v[pl.ds(s * D + 0, 16)] = a0 * inv
            ostage_v[pl.ds(s * D + 16, 16)] = a1 * inv
            ostage_v[pl.ds(s * D + 32, 16)] = a2 * inv
            ostage_v[pl.ds(s * D + 48, 16)] = a3 * inv

        for acc, _ in accs_sems:
            zero(acc)

        def blk_body(blk, carry):
            sbase = base + blk * SB
            pltpu.sync_copy(x_hbm.at[pl.ds(sbase * HIST, SB * HIST)], idx_v)
            for u, (acc, sem) in enumerate(accs_sems):
                fire(u, acc, sem)

            def quad_body(q, carry2):
                for u, (acc, sem) in enumerate(accs_sems):
                    wait_all(acc, sem)
                    reduce_and_rezero(4 * q + u, acc)

                    @pl.when(q < SB // 4 - 1)
                    def _():
                        fire(4 * q + 4 + u, acc, sem)

                return carry2

            lax.fori_loop(0, SB // 4, quad_body, 0)
            pltpu.sync_copy(ostage_v, out_hbm.at[pl.ds(sbase * D, SB * D)])
            return carry

        lax.fori_loop(0, S_PER_W // SB, blk_body, 0)

    return pool(x1, emb_lin).reshape(BATCH, D)


def _mlp_body(p_ref, w1_ref, b1_ref, w2_ref, b2_ref, o_ref):
    h = jnp.tanh(
        jnp.dot(p_ref[...], w1_ref[...], preferred_element_type=jnp.float32)
        + b1_ref[...])
    o_ref[...] = jnp.tanh(
        jnp.dot(h, w2_ref[...], preferred_element_type=jnp.float32)
        + b2_ref[...])


def _mlp_tc(pooled, W1, b1, W2, b2):
    MB = 2048
    return pl.pallas_call(
        _mlp_body,
        grid=(BATCH // MB,),
        in_specs=[
            pl.BlockSpec((MB, D), lambda i: (i, 0)),
            pl.BlockSpec((D, HIDDEN), lambda i: (0, 0)),
            pl.BlockSpec((1, HIDDEN), lambda i: (0, 0)),
            pl.BlockSpec((HIDDEN, OUT_D), lambda i: (0, 0)),
            pl.BlockSpec((1, OUT_D), lambda i: (0, 0)),
        ],
        out_specs=pl.BlockSpec((MB, OUT_D), lambda i: (i, 0)),
        out_shape=jax.ShapeDtypeStruct((BATCH, OUT_D), jnp.float32),
    )(pooled, W1.T, b1[None, :], W2.T, b2[None, :])


def kernel(x, emb, W1, b1, W2, b2):
    emb_lin = _detile_sc(emb).reshape(VOCAB, D)
    pooled = _pool_sc(x, emb_lin)
    return _mlp_tc(pooled, W1, b1, W2, b2)


# speedup vs baseline: 1.5394x; 1.5394x over previous
import functools

import jax
import jax.numpy as jnp
from jax import lax
from jax.experimental import pallas as pl
from jax.experimental.pallas import tpu as pltpu
from jax.experimental.pallas import tpu_sc as plsc

VOCAB = 1000000
D = 64
HIDDEN = 84
OUT_D = 64
BATCH = 16384
HIST = 200

NC = 2
NS = 16
NW = NC * NS
S_PER_W = BATCH // NW
SB = 64
C = 8
T = HIST // C


def _pool_sc(x, emb):
    x1 = x.reshape(BATCH * HIST)
    mesh = plsc.VectorSubcoreMesh(core_axis_name="c", subcore_axis_name="s")

    @functools.partial(
        pl.kernel,
        mesh=mesh,
        compiler_params=pltpu.CompilerParams(use_tc_tiling_on_sc=False),
        out_type=jax.ShapeDtypeStruct((BATCH * D,), jnp.float32),
        scratch_types=[
            pltpu.VMEM((SB * HIST,), jnp.int32),
            pltpu.VMEM((C, D), jnp.float32),
            pltpu.VMEM((C, D), jnp.float32),
            pltpu.VMEM((C, D), jnp.float32),
            pltpu.VMEM((C, D), jnp.float32),
            pltpu.VMEM((C, D), jnp.float32),
            pltpu.VMEM((C, D), jnp.float32),
            pltpu.VMEM((C, D), jnp.float32),
            pltpu.VMEM((C, D), jnp.float32),
            pltpu.VMEM((SB * D,), jnp.float32),
            pltpu.SemaphoreType.DMA,
            pltpu.SemaphoreType.DMA,
            pltpu.SemaphoreType.DMA,
            pltpu.SemaphoreType.DMA,
            pltpu.SemaphoreType.DMA,
            pltpu.SemaphoreType.DMA,
            pltpu.SemaphoreType.DMA,
            pltpu.SemaphoreType.DMA,
        ],
    )
    def pool(x_hbm, emb_hbm, out_hbm, idx_v, acc0, acc1, acc2, acc3,
             acc4, acc5, acc6, acc7,
             ostage_v, sem0, sem1, sem2, sem3, sem4, sem5, sem6, sem7):
        accs_sems = ((acc0, sem0), (acc1, sem1), (acc2, sem2), (acc3, sem3),
                     (acc4, sem4), (acc5, sem5), (acc6, sem6), (acc7, sem7))
        wid = lax.axis_index("s") * NC + lax.axis_index("c")
        base = wid * S_PER_W
        inv = jnp.float32(1.0 / HIST)
        z = jnp.zeros((16,), jnp.float32)

        def zero(acc):
            def zbody(j, carry):
                for u in range(2):
                    acc[2 * j + u, pl.ds(0, 16)] = z
                    acc[2 * j + u, pl.ds(16, 16)] = z
                    acc[2 * j + u, pl.ds(32, 16)] = z
                    acc[2 * j + u, pl.ds(48, 16)] = z
                return carry
            lax.fori_loop(0, C // 2, zbody, 0)

        def fire(s, acc, sem):
            for t in range(T):
                pltpu.async_copy(
                    emb_hbm.at[idx_v.at[pl.ds(s * HIST + t * C, C)]],
                    acc, sem, add=True)

        def wait_all(acc, sem):
            cp = pltpu.make_async_copy(
                emb_hbm.at[idx_v.at[pl.ds(0, C)]], acc, sem)
            for t in range(T):
                cp.wait()

        def reduce_and_rezero(s, acc):
            def rbody(j, accs):
                a0, a1, a2, a3 = accs
                for u in range(2):
                    a0 = a0 + acc[2 * j + u, pl.ds(0, 16)]
                    a1 = a1 + acc[2 * j + u, pl.ds(16, 16)]
                    a2 = a2 + acc[2 * j + u, pl.ds(32, 16)]
                    a3 = a3 + acc[2 * j + u, pl.ds(48, 16)]
                    acc[2 * j + u, pl.ds(0, 16)] = z
                    acc[2 * j + u, pl.ds(16, 16)] = z
                    acc[2 * j + u, pl.ds(32, 16)] = z
                    acc[2 * j + u, pl.ds(48, 16)] = z
                return (a0, a1, a2, a3)
            a0, a1, a2, a3 = lax.fori_loop(0, C // 2, rbody, (z, z, z, z))
            ostage_v[pl.ds(s * D + 0, 16)] = a0 * inv
            ostage_v[pl.ds(s * D + 16, 16)] = a1 * inv
            ostage_v[pl.ds(s * D + 32, 16)] = a2 * inv
            ostage_v[pl.ds(s * D + 48, 16)] = a3 * inv

        for acc, _ in accs_sems:
            zero(acc)

        def blk_body(blk, carry):
            sbase = base + blk * SB
            pltpu.sync_copy(x_hbm.at[pl.ds(sbase * HIST, SB * HIST)], idx_v)
            for u, (acc, sem) in enumerate(accs_sems):
                fire(u, acc, sem)

            def oct_body(q, carry2):
                for u, (acc, sem) in enumerate(accs_sems):
                    wait_all(acc, sem)
                    reduce_and_rezero(8 * q + u, acc)

                    @pl.when(q < SB // 8 - 1)
                    def _():
                        fire(8 * q + 8 + u, acc, sem)

                return carry2

            lax.fori_loop(0, SB // 8, oct_body, 0)
            pltpu.sync_copy(ostage_v, out_hbm.at[pl.ds(sbase * D, SB * D)])
            return carry

        lax.fori_loop(0, S_PER_W // SB, blk_body, 0)

    return pool(x1, emb).reshape(BATCH, D)


def _mlp_body(p_ref, w1_ref, b1_ref, w2_ref, b2_ref, o_ref):
    h = jnp.tanh(
        jnp.dot(p_ref[...], w1_ref[...], preferred_element_type=jnp.float32)
        + b1_ref[...])
    o_ref[...] = jnp.tanh(
        jnp.dot(h, w2_ref[...], preferred_element_type=jnp.float32)
        + b2_ref[...])


def _mlp_tc(pooled, W1, b1, W2, b2):
    MB = 2048
    return pl.pallas_call(
        _mlp_body,
        grid=(BATCH // MB,),
        in_specs=[
            pl.BlockSpec((MB, D), lambda i: (i, 0)),
            pl.BlockSpec((D, HIDDEN), lambda i: (0, 0)),
            pl.BlockSpec((1, HIDDEN), lambda i: (0, 0)),
            pl.BlockSpec((HIDDEN, OUT_D), lambda i: (0, 0)),
            pl.BlockSpec((1, OUT_D), lambda i: (0, 0)),
        ],
        out_specs=pl.BlockSpec((MB, OUT_D), lambda i: (i, 0)),
        out_shape=jax.ShapeDtypeStruct((BATCH, OUT_D), jnp.float32),
    )(pooled, W1.T, b1[None, :], W2.T, b2[None, :])


def kernel(x, emb, W1, b1, W2, b2):
    pooled = _pool_sc(x, emb)
    return _mlp_tc(pooled, W1, b1, W2, b2)
